# Initial kernel scaffold; baseline (speedup 1.0000x reference)
import jax
import jax.numpy as jnp
from jax.experimental import pallas as pl


def kernel(input_ids, base_weight, prior_matrix, gate_logits):
    raise NotImplementedError("write your pallas kernel here")


# speedup vs baseline: 1.9827x; 1.9827x over previous
import functools

import jax
import jax.numpy as jnp
from jax import lax
from jax.experimental import pallas as pl
from jax.experimental.pallas import tpu as pltpu
from jax.experimental.pallas import tpu_sc as plsc

D = 32
NC = 2
NS = 16
NW = NC * NS
N = 128
LANES = 16


@functools.cache
def _sc_call(bt: int):
    pw = bt // NW
    nch = pw // N
    assert pw % N == 0 and nch % 2 == 0

    mesh = plsc.VectorSubcoreMesh(core_axis_name="c", subcore_axis_name="s")

    @functools.partial(
        pl.kernel,
        mesh=mesh,
        compiler_params=pltpu.CompilerParams(use_tc_tiling_on_sc=False),
        out_type=jax.ShapeDtypeStruct((bt, D), jnp.float32),
        scratch_types=[
            pltpu.VMEM((nch, N), jnp.int32),
            *[pltpu.VMEM((N, D), jnp.float32) for _ in range(8)],
            *[pltpu.SemaphoreType.DMA for _ in range(8)],
        ],
    )
    def k(ids_hbm, base_hbm, prior_hbm, gate_hbm, out_hbm,
          idx_v, b0, p0, g0, o0, b1, p1, g1, o1,
          sb0, sp0, sg0, so0, sb1, sp1, sg1, so1):
        cid = lax.axis_index("c")
        sid = lax.axis_index("s")
        wid = sid * NC + cid
        tok0 = wid * pw

        pltpu.sync_copy(ids_hbm.at[wid], idx_v)

        bufs = [
            (b0, p0, g0, o0, sb0, sp0, sg0, so0),
            (b1, p1, g1, o1, sb1, sp1, sg1, so1),
        ]

        def fire_gathers(g, s):
            b, p, gt, _, sb, sp, sg, _ = bufs[s]
            idx = idx_v.at[g]
            pltpu.async_copy(base_hbm.at[idx], b, sb)
            pltpu.async_copy(prior_hbm.at[idx], p, sp)
            pltpu.async_copy(gate_hbm.at[idx], gt, sg)

        def wait_gathers(g, s):
            b, p, gt, _, sb, sp, sg, _ = bufs[s]
            idx = idx_v.at[g]
            pltpu.make_async_copy(base_hbm.at[idx], b, sb).wait()
            pltpu.make_async_copy(prior_hbm.at[idx], p, sp).wait()
            pltpu.make_async_copy(gate_hbm.at[idx], gt, sg).wait()

        def out_slice(g):
            return out_hbm.at[pl.ds(tok0 + g * N, N)]

        def wait_out(g, s):
            o, so = bufs[s][3], bufs[s][7]
            pltpu.make_async_copy(o, out_slice(g), so).wait()

        fire_gathers(0, 0)
        fire_gathers(1, 1)

        def body(gg, carry):
            for s in range(2):
                g = 2 * gg + s
                b, p, gt, o, sb, sp, sg, so = bufs[s]
                wait_gathers(g, s)

                @pl.when(gg > 0)
                def _():
                    wait_out(g - 2, s)

                def row(i, c):
                    for h in range(2):
                        sl = pl.ds(h * LANES, LANES)
                        gv = gt[i, sl]
                        w = 1.0 / (1.0 + jnp.exp(-gv))
                        o[i, sl] = b[i, sl] + w * p[i, sl]
                    return c

                lax.fori_loop(0, N, row, 0)
                pltpu.async_copy(o, out_slice(g), so)

                @pl.when(gg < (nch // 2 - 1))
                def _():
                    fire_gathers(g + 2, s)
            return carry

        lax.fori_loop(0, nch // 2, body, 0)
        wait_out(nch - 2, 0)
        wait_out(nch - 1, 1)

    return k


def kernel(input_ids, base_weight, prior_matrix, gate_logits):
    b, t = input_ids.shape
    bt = b * t
    ids = input_ids.reshape(NW, bt // (NW * N), N).astype(jnp.int32)
    out = _sc_call(bt)(ids, base_weight, prior_matrix, gate_logits)
    return out.reshape(b, t, D)
